# Initial kernel scaffold; baseline (speedup 1.0000x reference)
import jax
import jax.numpy as jnp
from jax.experimental import pallas as pl


def kernel(x, y, idx_i, idx_j):
    raise NotImplementedError("write your pallas kernel here")


# speedup vs baseline: 77.6373x; 77.6373x over previous
import functools

import jax
import jax.numpy as jnp
from jax import lax
from jax.experimental import pallas as pl
from jax.experimental.pallas import tpu as pltpu
from jax.experimental.pallas import tpu_sc as plsc

NC = 2
NS = 16
NW = NC * NS
LANES = 16

CHUNK = 2048
GATHER = 128


def _build(qp: int):
    n_chunks = qp // (NW * CHUNK)
    mesh = plsc.VectorSubcoreMesh(core_axis_name="c", subcore_axis_name="s")

    @functools.partial(
        pl.kernel,
        mesh=mesh,
        out_type=jax.ShapeDtypeStruct((qp,), jnp.float32),
        scratch_types=[
            pltpu.VMEM((CHUNK,), jnp.int32),
            pltpu.VMEM((CHUNK,), jnp.int32),
            pltpu.VMEM((CHUNK,), jnp.float32),
            pltpu.VMEM((CHUNK,), jnp.float32),
            pltpu.VMEM((CHUNK,), jnp.float32),
            pltpu.SemaphoreType.DMA,
        ],
    )
    def sc_kernel(x_hbm, y_hbm, ii_hbm, jj_hbm, out_hbm,
                  ii_v, jj_v, gx_v, gy_v, o_v, sem):
        wid = lax.axis_index("s") * NC + lax.axis_index("c")

        def chunk_body(ci, _):
            base = (wid * n_chunks + ci) * CHUNK
            pltpu.sync_copy(ii_hbm.at[pl.ds(base, CHUNK)], ii_v)
            pltpu.sync_copy(jj_hbm.at[pl.ds(base, CHUNK)], jj_v)
            copies = []
            for t in range(CHUNK // GATHER):
                sl = pl.ds(t * GATHER, GATHER)
                copies.append(pltpu.async_copy(
                    x_hbm.at[ii_v.at[sl]], gx_v.at[sl], sem))
                copies.append(pltpu.async_copy(
                    y_hbm.at[jj_v.at[sl]], gy_v.at[sl], sem))
            for cp in copies:
                cp.wait()

            def mul_body(k, _):
                s = pl.ds(k * LANES, LANES)
                o_v[s] = gx_v[s] * gy_v[s]
                return ()

            lax.fori_loop(0, CHUNK // LANES, mul_body, ())
            pltpu.sync_copy(o_v, out_hbm.at[pl.ds(base, CHUNK)])
            return ()

        lax.fori_loop(0, n_chunks, chunk_body, ())

    return sc_kernel


def kernel(x, y, idx_i, idx_j):
    q = idx_i.shape[0]
    step = NW * CHUNK
    qp = ((q + step - 1) // step) * step
    pad = qp - q
    if pad:
        zeros = jnp.zeros((pad,), jnp.int32)
        ii = jnp.concatenate([idx_i, zeros])
        jj = jnp.concatenate([idx_j, zeros])
    else:
        ii, jj = idx_i, idx_j
    out = _build(qp)(x, y, ii, jj)
    return out[:q]
